# Initial kernel scaffold; baseline (speedup 1.0000x reference)
import jax
import jax.numpy as jnp
from jax.experimental import pallas as pl


def kernel(X, Mu, Var, edge_index, W, S, lin_l_w, lin_l_b, lin_r_w, lin_r_b, att):
    raise NotImplementedError("write your pallas kernel here")


# speedup vs baseline: 10.9481x; 10.9481x over previous
import functools

import jax
import jax.numpy as jnp
from jax import lax
from jax.experimental import pallas as pl
from jax.experimental.pallas import tpu as pltpu
from jax.experimental.pallas import tpu_sc as plsc

N, C, G, E, H = 10000, 32, 128, 320000, 8
NPAD = 10240
EPB = 128
ROWS = 2560
EPAD = ROWS * EPB
NTILES = 32
TPB = ROWS // NTILES

_mesh = plsc.VectorSubcoreMesh(core_axis_name="c", subcore_axis_name="s")
_dn = (((1,), (1,)), ((), ()))


def _tc_proj_body(x_ref, wl_ref, bl_ref, wr_ref, br_ref, xl_ref, xr_ref):
    x = x_ref[...]
    xl_ref[...] = lax.dot_general(x, wl_ref[...], _dn,
                                  preferred_element_type=jnp.float32) + bl_ref[...]
    xr_ref[...] = lax.dot_general(x, wr_ref[...], _dn,
                                  preferred_element_type=jnp.float32) + br_ref[...]


_tc_proj = pl.pallas_call(
    _tc_proj_body,
    out_shape=[jax.ShapeDtypeStruct((N, H), jnp.float32),
               jax.ShapeDtypeStruct((N, H), jnp.float32)],
)


def _tc_dense_body(w_ref, x_ref, mu_ref, var_ref, s_ref, p_ref, q_ref, ll_ref):
    w = w_ref[...]
    m = jnp.max(w, axis=1, keepdims=True)
    e = jnp.exp(w - m)
    p = e / jnp.sum(e, axis=1, keepdims=True)
    p_ref[...] = p
    q_ref[...] = jnp.log(p + 1e-8)
    iv = 1.0 / var_ref[...]
    x = x_ref[...]
    xv = x * iv
    A = lax.dot_general(xv, mu_ref[...], _dn,
                        preferred_element_type=jnp.float32)
    a = jnp.sum(x * xv, axis=1, keepdims=True)
    mu2 = mu_ref[...] * mu_ref[...]
    qrow = lax.dot_general(iv, mu2, _dn,
                           preferred_element_type=jnp.float32)
    s = s_ref[...]
    F = -0.5 * (a - 2.0 * s * A + (s * s) * qrow)
    ll_ref[...] = (jnp.sum(p * F) * (1.0 / N)).reshape(1, 1)


_tc_dense = pl.pallas_call(
    _tc_dense_body,
    out_shape=[jax.ShapeDtypeStruct((N, C), jnp.float32),
               jax.ShapeDtypeStruct((N, C), jnp.float32),
               jax.ShapeDtypeStruct((1, 1), jnp.float32)],
)


@functools.partial(
    pl.kernel,
    out_type=[jax.ShapeDtypeStruct((ROWS, EPB), jnp.float32),
              jax.ShapeDtypeStruct((2, NPAD), jnp.float32)],
    mesh=_mesh,
    compiler_params=pltpu.CompilerParams(needs_layout_passes=False, use_tc_tiling_on_sc=False),
    scratch_types=[
        pltpu.VMEM((TPB, EPB), jnp.int32),
        pltpu.VMEM((TPB, EPB), jnp.int32),
        pltpu.VMEM((TPB, EPB), jnp.float32),
        pltpu.VMEM((EPB, H), jnp.float32),
        pltpu.VMEM((EPB, H), jnp.float32),
        pltpu.VMEM((EPB, H), jnp.float32),
        pltpu.VMEM((EPB, H), jnp.float32),
        pltpu.VMEM((16,), jnp.float32),
        pltpu.VMEM((NPAD // 16,), jnp.float32),
        pltpu.VMEM_SHARED((NPAD,), jnp.float32),
        pltpu.SemaphoreType.DMA,
        pltpu.SemaphoreType.DMA,
    ],
)
def _sc_edge1(src_hbm, dst_hbm, xl_hbm, xr_hbm, att_hbm, ex_hbm, spart_hbm,
              src_v, dst_v, ex_v, xla, xra, xlb, xrb, att_v, zbuf, s_sh,
              sem_a, sem_b):
    cid = lax.axis_index("c")
    sid = lax.axis_index("s")
    wid = cid * 16 + sid
    base = wid * TPB
    nsub = NPAD // 16

    pltpu.sync_copy(src_hbm.at[pl.ds(base, TPB)], src_v)
    pltpu.sync_copy(dst_hbm.at[pl.ds(base, TPB)], dst_v)
    pltpu.sync_copy(att_hbm, att_v)

    iota = lax.iota(jnp.int32, 16)
    zero16 = jnp.zeros((16,), jnp.float32)

    def _zero(i, carry):
        zbuf[pl.ds(i * 16, 16)] = zero16
        return carry

    lax.fori_loop(0, nsub // 16, _zero, 0)
    pltpu.sync_copy(zbuf, s_sh.at[pl.ds(sid * nsub, nsub)])
    plsc.subcore_barrier()

    def _fire(b, xs, rs, sem):
        pltpu.async_copy(xl_hbm.at[src_v.at[b]], xs, sem)
        pltpu.async_copy(xr_hbm.at[dst_v.at[b]], rs, sem)

    def _drain(xs, rs, sem):
        pltpu.make_async_copy(xl_hbm.at[pl.ds(0, EPB)], xs, sem).wait()
        pltpu.make_async_copy(xl_hbm.at[pl.ds(0, EPB)], rs, sem).wait()

    def _compute(b, xs, rs):
        grow = base + b
        att_full = att_v[...]
        for k in range(EPB // 16):
            rows = iota + (k * 16)
            acc_a = jnp.zeros((16,), jnp.float32)
            acc_b = jnp.zeros((16,), jnp.float32)
            for h in range(H):
                hsp = jnp.full((16,), h, jnp.int32)
                av = att_full[h]
                v = plsc.load_gather(xs, [rows, hsp]) + plsc.load_gather(rs, [rows, hsp])
                acc_a = acc_a + av * v
                acc_b = acc_b + av * jnp.abs(v)
            exv = jnp.exp(0.6 * acc_a + 0.4 * acc_b)
            ids = iota + (grow * EPB + k * 16)
            exv = jnp.where(ids < E, exv, 0.0)
            ex_v[b, pl.ds(k * 16, 16)] = exv
        pltpu.sync_copy(ex_v.at[b], s_sh.at[dst_v.at[b]], add=True)

    _fire(0, xla, xra, sem_a)

    def _loop(g, carry):
        b0 = 2 * g
        _fire(b0 + 1, xlb, xrb, sem_b)
        _drain(xla, xra, sem_a)
        _compute(b0, xla, xra)

        @pl.when(g < TPB // 2 - 1)
        def _():
            _fire(b0 + 2, xla, xra, sem_a)

        _drain(xlb, xrb, sem_b)
        _compute(b0 + 1, xlb, xrb)
        return carry

    lax.fori_loop(0, TPB // 2, _loop, 0)

    pltpu.sync_copy(ex_v, ex_hbm.at[pl.ds(base, TPB)])
    plsc.subcore_barrier()
    pltpu.sync_copy(s_sh.at[pl.ds(sid * nsub, nsub)],
                    spart_hbm.at[cid, pl.ds(sid * nsub, nsub)])


@functools.partial(
    pl.kernel,
    out_type=jax.ShapeDtypeStruct((NTILES, 16), jnp.float32),
    mesh=_mesh,
    compiler_params=pltpu.CompilerParams(needs_layout_passes=False, use_tc_tiling_on_sc=False),
    scratch_types=[
        pltpu.VMEM((TPB, EPB), jnp.int32),
        pltpu.VMEM((TPB, EPB), jnp.int32),
        pltpu.VMEM((TPB, EPB), jnp.float32),
        pltpu.VMEM((NPAD,), jnp.float32),
        pltpu.VMEM((NPAD,), jnp.float32),
        pltpu.VMEM((EPB, C), jnp.float32),
        pltpu.VMEM((EPB, C), jnp.float32),
        pltpu.VMEM((EPB, C), jnp.float32),
        pltpu.VMEM((EPB, C), jnp.float32),
        pltpu.VMEM((16,), jnp.float32),
        pltpu.SemaphoreType.DMA,
        pltpu.SemaphoreType.DMA,
    ],
)
def _sc_edge2(src_hbm, dst_hbm, ex_hbm, spart_hbm, p_hbm, q_hbm, out_hbm,
              src_v, dst_v, ex_v, s_v, st_v, pa, qa, pb, qb, orow,
              sem_a, sem_b):
    cid = lax.axis_index("c")
    sid = lax.axis_index("s")
    wid = cid * 16 + sid
    base = wid * TPB

    pltpu.sync_copy(src_hbm.at[pl.ds(base, TPB)], src_v)
    pltpu.sync_copy(dst_hbm.at[pl.ds(base, TPB)], dst_v)
    pltpu.sync_copy(ex_hbm.at[pl.ds(base, TPB)], ex_v)
    pltpu.sync_copy(spart_hbm.at[0], s_v)
    pltpu.sync_copy(spart_hbm.at[1], st_v)

    iota = lax.iota(jnp.int32, 16)

    def _sum(i, carry):
        sl = pl.ds(i * 16, 16)
        s_v[sl] = s_v[sl] + st_v[sl] + 1e-16
        return carry

    lax.fori_loop(0, NPAD // 16, _sum, 0)

    def _fire(b, ps, qs, sem):
        pltpu.async_copy(p_hbm.at[src_v.at[b]], ps, sem)
        pltpu.async_copy(q_hbm.at[dst_v.at[b]], qs, sem)

    def _drain(ps, qs, sem):
        pltpu.make_async_copy(p_hbm.at[pl.ds(0, EPB)], ps, sem).wait()
        pltpu.make_async_copy(p_hbm.at[pl.ds(0, EPB)], qs, sem).wait()

    def _compute(b, ps, qs, acc):
        for k in range(EPB // 16):
            sl = pl.ds(k * 16, 16)
            rows = iota + (k * 16)
            sv = plsc.load_gather(s_v, [dst_v[b, sl]])
            alpha = ex_v[b, sl] / sv
            d = [jnp.zeros((16,), jnp.float32) for _ in range(4)]
            for c in range(C):
                csp = jnp.full((16,), c, jnp.int32)
                pc = plsc.load_gather(ps, [rows, csp])
                qc = plsc.load_gather(qs, [rows, csp])
                d[c % 4] = d[c % 4] + pc * qc
            acc = acc + alpha * ((d[0] + d[1]) + (d[2] + d[3]))
        return acc

    _fire(0, pa, qa, sem_a)

    def _loop(g, acc):
        b0 = 2 * g
        _fire(b0 + 1, pb, qb, sem_b)
        _drain(pa, qa, sem_a)
        acc = _compute(b0, pa, qa, acc)

        @pl.when(g < TPB // 2 - 1)
        def _():
            _fire(b0 + 2, pa, qa, sem_a)

        _drain(pb, qb, sem_b)
        acc = _compute(b0 + 1, pb, qb, acc)
        return acc

    acc = lax.fori_loop(0, TPB // 2, _loop, jnp.zeros((16,), jnp.float32))
    orow[...] = acc * (-1.0 / N)
    pltpu.sync_copy(orow, out_hbm.at[wid])


def kernel(X, Mu, Var, edge_index, W, S, lin_l_w, lin_l_b, lin_r_w, lin_r_b, att):
    xl, xr = _tc_proj(X, lin_l_w, lin_l_b.reshape(1, H), lin_r_w,
                      lin_r_b.reshape(1, H))
    P, Q, ll = _tc_dense(W, X, Mu, Var.reshape(1, G), S)
    pad = EPAD - E
    src2 = jnp.concatenate(
        [edge_index[0], jnp.zeros((pad,), jnp.int32)]).reshape(ROWS, EPB)
    dst2 = jnp.concatenate(
        [edge_index[1], jnp.zeros((pad,), jnp.int32)]).reshape(ROWS, EPB)
    att16 = jnp.pad(att, (0, 16 - H))
    ex, spart = _sc_edge1(src2, dst2, xl, xr, att16)
    ce_part = _sc_edge2(src2, dst2, ex, spart, P, Q)
    return (ll[0, 0], jnp.sum(ce_part), P)


# speedup vs baseline: 14.7174x; 1.3443x over previous
import functools

import jax
import jax.numpy as jnp
from jax import lax
from jax.experimental import pallas as pl
from jax.experimental.pallas import tpu as pltpu
from jax.experimental.pallas import tpu_sc as plsc

N, C, G, E, H = 10000, 32, 128, 320000, 8
NPAD = 10240
EPB = 128
ROWS = 2560
EPAD = ROWS * EPB
NTILES = 32
TPB = ROWS // NTILES

_mesh = plsc.VectorSubcoreMesh(core_axis_name="c", subcore_axis_name="s")
_dn = (((1,), (1,)), ((), ()))


def _tc_proj_body(x_ref, wl_ref, bl_ref, wr_ref, br_ref, xl_ref, xr_ref):
    x = x_ref[...]
    xl_ref[...] = lax.dot_general(x, wl_ref[...], _dn,
                                  preferred_element_type=jnp.float32) + bl_ref[...]
    xr_ref[...] = lax.dot_general(x, wr_ref[...], _dn,
                                  preferred_element_type=jnp.float32) + br_ref[...]


_tc_proj = pl.pallas_call(
    _tc_proj_body,
    out_shape=[jax.ShapeDtypeStruct((N, H), jnp.float32),
               jax.ShapeDtypeStruct((N, H), jnp.float32)],
)


def _tc_dense_body(w_ref, x_ref, mu_ref, var_ref, s_ref, p_ref, q_ref, ll_ref):
    w = w_ref[...]
    m = jnp.max(w, axis=1, keepdims=True)
    e = jnp.exp(w - m)
    p = e / jnp.sum(e, axis=1, keepdims=True)
    p_ref[...] = p
    q_ref[...] = jnp.log(p + 1e-8)
    iv = 1.0 / var_ref[...]
    x = x_ref[...]
    xv = x * iv
    A = lax.dot_general(xv, mu_ref[...], _dn,
                        preferred_element_type=jnp.float32)
    a = jnp.sum(x * xv, axis=1, keepdims=True)
    mu2 = mu_ref[...] * mu_ref[...]
    qrow = lax.dot_general(iv, mu2, _dn,
                           preferred_element_type=jnp.float32)
    s = s_ref[...]
    F = -0.5 * (a - 2.0 * s * A + (s * s) * qrow)
    ll_ref[...] = (jnp.sum(p * F) * (1.0 / N)).reshape(1, 1)


_tc_dense = pl.pallas_call(
    _tc_dense_body,
    out_shape=[jax.ShapeDtypeStruct((N, C), jnp.float32),
               jax.ShapeDtypeStruct((N, C), jnp.float32),
               jax.ShapeDtypeStruct((1, 1), jnp.float32)],
)


@functools.partial(
    pl.kernel,
    out_type=[jax.ShapeDtypeStruct((ROWS, EPB), jnp.float32),
              jax.ShapeDtypeStruct((2, NPAD), jnp.float32)],
    mesh=_mesh,
    compiler_params=pltpu.CompilerParams(needs_layout_passes=False, use_tc_tiling_on_sc=False),
    scratch_types=[
        pltpu.VMEM((TPB, EPB), jnp.int32),
        pltpu.VMEM((TPB, EPB), jnp.int32),
        pltpu.VMEM((TPB, EPB), jnp.float32),
        [pltpu.VMEM((EPB, H), jnp.float32) for _ in range(8)],
        [pltpu.VMEM((EPB, H), jnp.float32) for _ in range(8)],
        pltpu.VMEM((16,), jnp.float32),
        pltpu.VMEM((NPAD // 16,), jnp.float32),
        pltpu.VMEM_SHARED((NPAD,), jnp.float32),
        [pltpu.SemaphoreType.DMA for _ in range(8)],
    ],
)
def _sc_edge1(src_hbm, dst_hbm, xl_hbm, xr_hbm, att_hbm, ex_hbm, spart_hbm,
              src_v, dst_v, ex_v, xls, xrs, att_v, zbuf, s_sh, sems):
    cid = lax.axis_index("c")
    sid = lax.axis_index("s")
    wid = cid * 16 + sid
    base = wid * TPB
    nsub = NPAD // 16

    pltpu.sync_copy(src_hbm.at[pl.ds(base, TPB)], src_v)
    pltpu.sync_copy(dst_hbm.at[pl.ds(base, TPB)], dst_v)
    pltpu.sync_copy(att_hbm, att_v)

    iota = lax.iota(jnp.int32, 16)
    zero16 = jnp.zeros((16,), jnp.float32)

    def _zero(i, carry):
        zbuf[pl.ds(i * 16, 16)] = zero16
        return carry

    lax.fori_loop(0, nsub // 16, _zero, 0)
    pltpu.sync_copy(zbuf, s_sh.at[pl.ds(sid * nsub, nsub)])
    plsc.subcore_barrier()

    def _fire(b, xs, rs, sem):
        pltpu.async_copy(xl_hbm.at[src_v.at[b]], xs, sem)
        pltpu.async_copy(xr_hbm.at[dst_v.at[b]], rs, sem)

    def _drain(xs, rs, sem):
        pltpu.make_async_copy(xl_hbm.at[pl.ds(0, EPB)], xs, sem).wait()
        pltpu.make_async_copy(xl_hbm.at[pl.ds(0, EPB)], rs, sem).wait()

    def _compute(b, xs, rs):
        grow = base + b
        att_full = att_v[...]

        def _group(k, carry):
            rows = iota + k * 16
            acc_a = jnp.zeros((16,), jnp.float32)
            acc_b = jnp.zeros((16,), jnp.float32)
            for h in range(H):
                hsp = jnp.full((16,), h, jnp.int32)
                av = att_full[h]
                v = plsc.load_gather(xs, [rows, hsp]) + plsc.load_gather(rs, [rows, hsp])
                acc_a = acc_a + av * v
                acc_b = acc_b + av * jnp.abs(v)
            exv = jnp.exp(0.6 * acc_a + 0.4 * acc_b)
            ids = iota + (grow * EPB + k * 16)
            exv = jnp.where(ids < E, exv, 0.0)
            ex_v[b, pl.ds(k * 16, 16)] = exv
            return carry

        lax.fori_loop(0, EPB // 16, _group, 0)
        pltpu.sync_copy(ex_v.at[b], s_sh.at[dst_v.at[b]], add=True)

    for j in range(7):
        _fire(j, xls[j], xrs[j], sems[j])

    def _loop(g, carry):
        for j in range(8):
            b = 8 * g + j
            jf = (j + 7) % 8

            @pl.when(b + 7 < TPB)
            def _():
                _fire(b + 7, xls[jf], xrs[jf], sems[jf])

            _drain(xls[j], xrs[j], sems[j])
            _compute(b, xls[j], xrs[j])
        return carry

    lax.fori_loop(0, TPB // 8, _loop, 0)

    pltpu.sync_copy(ex_v, ex_hbm.at[pl.ds(base, TPB)])
    plsc.subcore_barrier()
    pltpu.sync_copy(s_sh.at[pl.ds(sid * nsub, nsub)],
                    spart_hbm.at[cid, pl.ds(sid * nsub, nsub)])


@functools.partial(
    pl.kernel,
    out_type=jax.ShapeDtypeStruct((NTILES, 16), jnp.float32),
    mesh=_mesh,
    compiler_params=pltpu.CompilerParams(needs_layout_passes=False, use_tc_tiling_on_sc=False),
    scratch_types=[
        pltpu.VMEM((TPB, EPB), jnp.int32),
        pltpu.VMEM((TPB, EPB), jnp.int32),
        pltpu.VMEM((TPB, EPB), jnp.float32),
        pltpu.VMEM((NPAD,), jnp.float32),
        pltpu.VMEM((NPAD,), jnp.float32),
        [pltpu.VMEM((EPB, C), jnp.float32) for _ in range(5)],
        [pltpu.VMEM((EPB, C), jnp.float32) for _ in range(5)],
        pltpu.VMEM((16,), jnp.float32),
        [pltpu.SemaphoreType.DMA for _ in range(5)],
    ],
)
def _sc_edge2(src_hbm, dst_hbm, ex_hbm, spart_hbm, p_hbm, q_hbm, out_hbm,
              src_v, dst_v, ex_v, s_v, st_v, pss, qss, orow, sems):
    cid = lax.axis_index("c")
    sid = lax.axis_index("s")
    wid = cid * 16 + sid
    base = wid * TPB

    pltpu.sync_copy(src_hbm.at[pl.ds(base, TPB)], src_v)
    pltpu.sync_copy(dst_hbm.at[pl.ds(base, TPB)], dst_v)
    pltpu.sync_copy(ex_hbm.at[pl.ds(base, TPB)], ex_v)
    pltpu.sync_copy(spart_hbm.at[0], s_v)
    pltpu.sync_copy(spart_hbm.at[1], st_v)

    iota = lax.iota(jnp.int32, 16)

    def _sum(i, carry):
        sl = pl.ds(i * 16, 16)
        s_v[sl] = s_v[sl] + st_v[sl] + 1e-16
        return carry

    lax.fori_loop(0, NPAD // 16, _sum, 0)

    def _fire(b, ps, qs, sem):
        pltpu.async_copy(p_hbm.at[src_v.at[b]], ps, sem)
        pltpu.async_copy(q_hbm.at[dst_v.at[b]], qs, sem)

    def _drain(ps, qs, sem):
        pltpu.make_async_copy(p_hbm.at[pl.ds(0, EPB)], ps, sem).wait()
        pltpu.make_async_copy(p_hbm.at[pl.ds(0, EPB)], qs, sem).wait()

    def _compute(b, ps, qs, acc):
        def _group(k, acc):
            sl = pl.ds(k * 16, 16)
            rows = iota + k * 16
            sv = plsc.load_gather(s_v, [dst_v[b, sl]])
            alpha = ex_v[b, sl] / sv
            d = [jnp.zeros((16,), jnp.float32) for _ in range(4)]
            for c in range(C):
                csp = jnp.full((16,), c, jnp.int32)
                pc = plsc.load_gather(ps, [rows, csp])
                qc = plsc.load_gather(qs, [rows, csp])
                d[c % 4] = d[c % 4] + pc * qc
            return acc + alpha * ((d[0] + d[1]) + (d[2] + d[3]))

        return lax.fori_loop(0, EPB // 16, _group, acc)

    for j in range(4):
        _fire(j, pss[j], qss[j], sems[j])

    def _loop(g, acc):
        for j in range(5):
            b = 5 * g + j
            jf = (j + 4) % 5

            @pl.when(b + 4 < TPB)
            def _():
                _fire(b + 4, pss[jf], qss[jf], sems[jf])

            _drain(pss[j], qss[j], sems[j])
            acc = _compute(b, pss[j], qss[j], acc)
        return acc

    acc = lax.fori_loop(0, TPB // 5, _loop, jnp.zeros((16,), jnp.float32))
    orow[...] = acc * (-1.0 / N)
    pltpu.sync_copy(orow, out_hbm.at[wid])


def kernel(X, Mu, Var, edge_index, W, S, lin_l_w, lin_l_b, lin_r_w, lin_r_b, att):
    xl, xr = _tc_proj(X, lin_l_w, lin_l_b.reshape(1, H), lin_r_w,
                      lin_r_b.reshape(1, H))
    P, Q, ll = _tc_dense(W, X, Mu, Var.reshape(1, G), S)
    pad = EPAD - E
    src2 = jnp.concatenate(
        [edge_index[0], jnp.zeros((pad,), jnp.int32)]).reshape(ROWS, EPB)
    dst2 = jnp.concatenate(
        [edge_index[1], jnp.zeros((pad,), jnp.int32)]).reshape(ROWS, EPB)
    att16 = jnp.pad(att, (0, 16 - H))
    ex, spart = _sc_edge1(src2, dst2, xl, xr, att16)
    ce_part = _sc_edge2(src2, dst2, ex, spart, P, Q)
    return (ll[0, 0], jnp.sum(ce_part), P)
